# baseline (speedup 1.0000x reference)
import jax
import jax.numpy as jnp
from jax import lax
from jax.experimental import pallas as pl
from jax.experimental.pallas import tpu as pltpu
from jax.experimental.pallas import tpu_sc as plsc

NC = 2
NS = 16
NW = NC * NS

V = 1000000
D = 64
B = 4096
S = 200
NFULL = V // 128
TAIL = V - NFULL * 128
BASE_C = NFULL // NW

_mesh = plsc.VectorSubcoreMesh(
    core_axis_name="c", subcore_axis_name="s", num_cores=NC, num_subcores=NS
)
_params = pltpu.CompilerParams(use_tc_tiling_on_sc=True,
                               needs_layout_passes=False)


def _k1_body(vt_hbm, tail_hbm, t128_hbm, slab_v, tr_v, tail_v, sem):
    wid = lax.axis_index("s") * NC + lax.axis_index("c")
    n_c = BASE_C + jnp.where(wid < NFULL - BASE_C * NW, 1, 0)

    def step(ci, carry):
        c = ci * NW + wid
        pltpu.sync_copy(vt_hbm.at[:, pl.ds(c * 128, 128)], slab_v)

        def row(i, carry2):
            for k in range(4):
                col = jnp.full((16,), i, jnp.int32)
                rowi = lax.iota(jnp.int32, 16) + (16 * k)
                tr_v[i, pl.ds(16 * k, 16)] = plsc.load_gather(
                    slab_v, [rowi, col])
            return carry2

        lax.fori_loop(0, 128, row, 0, unroll=2)
        pltpu.sync_copy(tr_v, t128_hbm.at[pl.ds(c * 128, 128)])
        return carry

    lax.fori_loop(0, n_c, step, 0)

    @pl.when(wid == 0)
    def _tail():
        pltpu.sync_copy(tail_hbm, tail_v)
        pltpu.sync_copy(tail_v, t128_hbm.at[pl.ds(NFULL * 128, TAIL)])


_k1 = pl.kernel(
    _k1_body,
    out_type=jax.ShapeDtypeStruct((V, 128), jnp.float32),
    mesh=_mesh,
    scratch_types=[
        pltpu.VMEM((D, 128), jnp.float32),
        pltpu.VMEM((128, 128), jnp.float32),
        pltpu.VMEM((TAIL, 128), jnp.float32),
        pltpu.SemaphoreType.DMA,
    ],
    compiler_params=_params,
)


def _k2_body(idx_hbm, t128_hbm, out_hbm, idx_v, r0, r1, t0, t1,
             g0, g1, s0, s1):
    wid = lax.axis_index("s") * NC + lax.axis_index("c")
    bcol = wid * (B // NW)
    rows = [r0, r1]
    trans = [t0, t1]
    gsem = [g0, g1]
    ssem = [s0, s1]

    def sblock(t, carry):
        sbase = t * 8
        pltpu.sync_copy(idx_hbm.at[pl.ds(sbase, 8), pl.ds(bcol, 128)], idx_v)
        for si in range(8):
            b = si % 2
            s = sbase + si

            @pl.when(jnp.logical_or(t > 0, si >= 2))
            def _drain(b=b, s=s):
                pltpu.make_async_copy(
                    trans[b], out_hbm.at[s, :, pl.ds(bcol, 128)],
                    ssem[b]).wait()

            pltpu.async_copy(t128_hbm.at[idx_v.at[si]], rows[b],
                             gsem[b]).wait()

            def trow(i, carry2, b=b):
                for k in range(8):
                    rowi = lax.iota(jnp.int32, 16) + (16 * k)
                    col = jnp.full((16,), i, jnp.int32)
                    trans[b][i, pl.ds(16 * k, 16)] = plsc.load_gather(
                        rows[b], [rowi, col])
                return carry2

            lax.fori_loop(0, D, trow, 0, unroll=2)
            pltpu.async_copy(trans[b], out_hbm.at[s, :, pl.ds(bcol, 128)],
                             ssem[b])
        return carry

    lax.fori_loop(0, S // 8, sblock, 0)
    for b in range(2):
        s = S - 2 + b
        pltpu.make_async_copy(trans[b], out_hbm.at[s, :, pl.ds(bcol, 128)],
                              ssem[b]).wait()


_k2 = pl.kernel(
    _k2_body,
    out_type=jax.ShapeDtypeStruct((S, D, B), jnp.float32),
    mesh=_mesh,
    scratch_types=[
        pltpu.VMEM((8, B // NW), jnp.int32),
        pltpu.VMEM((B // NW, 128), jnp.float32),
        pltpu.VMEM((B // NW, 128), jnp.float32),
        pltpu.VMEM((D, 128), jnp.float32),
        pltpu.VMEM((D, 128), jnp.float32),
        pltpu.SemaphoreType.DMA,
        pltpu.SemaphoreType.DMA,
        pltpu.SemaphoreType.DMA,
        pltpu.SemaphoreType.DMA,
    ],
    compiler_params=_params,
)


def kernel(word_indices, vocabulary):
    idx = word_indices.astype(jnp.int32)
    tail = jnp.pad(lax.slice(vocabulary, (NFULL * 128, 0), (V, D)),
                   ((0, 0), (0, 128 - D)))
    t128 = _k1(vocabulary.T, tail)
    out_phys = _k2(idx.T, t128)
    return out_phys.transpose(2, 0, 1)


# speedup vs baseline: 1.1695x; 1.1695x over previous
import jax
import jax.numpy as jnp
from jax import lax
from jax.experimental import pallas as pl
from jax.experimental.pallas import tpu as pltpu
from jax.experimental.pallas import tpu_sc as plsc

NC = 2
NS = 16
NW = NC * NS

V = 1000000
D = 64
B = 4096
S = 200
NFULL = V // 128
TAIL = V - NFULL * 128
BASE_C = NFULL // NW

_mesh = plsc.VectorSubcoreMesh(
    core_axis_name="c", subcore_axis_name="s", num_cores=NC, num_subcores=NS
)
_params = pltpu.CompilerParams(use_tc_tiling_on_sc=True,
                               needs_layout_passes=False)


def _k1_body(vt_hbm, tail_hbm, t128_hbm, slab_v, tr_v, tail_v, sem):
    wid = lax.axis_index("s") * NC + lax.axis_index("c")
    n_c = BASE_C + jnp.where(wid < NFULL - BASE_C * NW, 1, 0)

    iotas = [lax.iota(jnp.int32, 16) + (16 * k) for k in range(8)]

    def step(ci, carry):
        c = ci * NW + wid
        pltpu.sync_copy(vt_hbm.at[:, pl.ds(c * 128, 128)], slab_v)

        def dblk(db, carry2):
            for dd in range(8):
                d = db * 8 + dd
                dfull = jnp.full((16,), d, jnp.int32)
                for k in range(8):
                    vals = slab_v[d, pl.ds(16 * k, 16)]
                    plsc.store_scatter(tr_v, [iotas[k], dfull], vals)
            return carry2

        lax.fori_loop(0, D // 8, dblk, 0)
        pltpu.sync_copy(tr_v, t128_hbm.at[pl.ds(c * 128, 128)])
        return carry

    lax.fori_loop(0, n_c, step, 0)

    @pl.when(wid == 0)
    def _tail():
        pltpu.sync_copy(tail_hbm, tail_v)
        pltpu.sync_copy(tail_v, t128_hbm.at[pl.ds(NFULL * 128, TAIL)])


_k1 = pl.kernel(
    _k1_body,
    out_type=jax.ShapeDtypeStruct((V, 128), jnp.float32),
    mesh=_mesh,
    scratch_types=[
        pltpu.VMEM((D, 128), jnp.float32),
        pltpu.VMEM((128, 128), jnp.float32),
        pltpu.VMEM((TAIL, 128), jnp.float32),
        pltpu.SemaphoreType.DMA,
    ],
    compiler_params=_params,
)


def _k2_body(idx_hbm, t128_hbm, out_hbm, idx_v, r0, r1, t0, t1,
             g0, g1, s0, s1):
    wid = lax.axis_index("s") * NC + lax.axis_index("c")
    bcol = wid * (B // NW)
    rows = [r0, r1]
    trans = [t0, t1]
    gsem = [g0, g1]
    ssem = [s0, s1]
    iotas = [lax.iota(jnp.int32, 16) + (16 * k) for k in range(4)]

    def sblock(t, carry):
        sbase = t * 8
        pltpu.sync_copy(idx_hbm.at[pl.ds(sbase, 8), pl.ds(bcol, 128)], idx_v)
        for si in range(8):
            b = si % 2
            s = sbase + si

            @pl.when(jnp.logical_or(t > 0, si >= 2))
            def _drain(b=b, s=s):
                pltpu.make_async_copy(
                    trans[b], out_hbm.at[s, :, pl.ds(bcol, 128)],
                    ssem[b]).wait()

            pltpu.async_copy(t128_hbm.at[idx_v.at[si]], rows[b],
                             gsem[b]).wait()
            def jblk(jb, carry2, b=b):
                for jj in range(16):
                    j = jb * 16 + jj
                    jfull = jnp.full((16,), j, jnp.int32)
                    for k in range(4):
                        vals = rows[b][j, pl.ds(16 * k, 16)]
                        plsc.store_scatter(trans[b], [iotas[k], jfull], vals)
                return carry2

            lax.fori_loop(0, (B // NW) // 16, jblk, 0)
            pltpu.async_copy(trans[b], out_hbm.at[s, :, pl.ds(bcol, 128)],
                             ssem[b])
        return carry

    lax.fori_loop(0, S // 8, sblock, 0)
    for b in range(2):
        s = S - 2 + b
        pltpu.make_async_copy(trans[b], out_hbm.at[s, :, pl.ds(bcol, 128)],
                              ssem[b]).wait()


_k2 = pl.kernel(
    _k2_body,
    out_type=jax.ShapeDtypeStruct((S, D, B), jnp.float32),
    mesh=_mesh,
    scratch_types=[
        pltpu.VMEM((8, B // NW), jnp.int32),
        pltpu.VMEM((B // NW, 128), jnp.float32),
        pltpu.VMEM((B // NW, 128), jnp.float32),
        pltpu.VMEM((D, 128), jnp.float32),
        pltpu.VMEM((D, 128), jnp.float32),
        pltpu.SemaphoreType.DMA,
        pltpu.SemaphoreType.DMA,
        pltpu.SemaphoreType.DMA,
        pltpu.SemaphoreType.DMA,
    ],
    compiler_params=_params,
)


def kernel(word_indices, vocabulary):
    idx = word_indices.astype(jnp.int32)
    tail = jnp.pad(lax.slice(vocabulary, (NFULL * 128, 0), (V, D)),
                   ((0, 0), (0, 128 - D)))
    t128 = _k1(vocabulary.T, tail)
    out_phys = _k2(idx.T, t128)
    return out_phys.transpose(2, 0, 1)


# speedup vs baseline: 2.7916x; 2.3870x over previous
import jax
import jax.numpy as jnp
from jax import lax
from jax.experimental import pallas as pl
from jax.experimental.pallas import tpu as pltpu
from jax.experimental.pallas import tpu_sc as plsc

NC = 2
NS = 16
NW = NC * NS

B = 4096
S = 200
D_MODEL = 64
ROWS_PER_W = B // NW
CB = 4
NBUF = 2
N_STEPS = ROWS_PER_W // CB
assert N_STEPS % NBUF == 0
SPLITS = [(0, 128), (128, S - 128)]


def _body(idx_hbm, table_hbm, out_hbm, i0, i1, r0, r1, g0, g1, s0, s1):
    wid = lax.axis_index("s") * NC + lax.axis_index("c")
    base = wid * ROWS_PER_W
    idx_v = [i0, i1]
    rows = [r0, r1]
    gsem = [g0, g1]
    ssem = [s0, s1]

    def outer(t, carry):
        gathers = []
        for b in range(NBUF):
            row = base + (t * NBUF + b) * CB

            @pl.when(t > 0)
            def _drain_store(b=b, row=row):
                pltpu.make_async_copy(rows[b], out_hbm.at[pl.ds(row, CB)],
                                      ssem[b]).wait()

            pltpu.sync_copy(idx_hbm.at[pl.ds(row, CB)], idx_v[b])
            gathers.append([
                pltpu.async_copy(table_hbm.at[idx_v[b].at[r, pl.ds(o, w)]],
                                 rows[b].at[r, pl.ds(o, w)], gsem[b])
                for r in range(CB)
                for (o, w) in SPLITS
            ])
        for b in range(NBUF):
            row = base + (t * NBUF + b) * CB
            for c in gathers[b]:
                c.wait()
            pltpu.async_copy(rows[b], out_hbm.at[pl.ds(row, CB)], ssem[b])
        return carry

    lax.fori_loop(0, N_STEPS // NBUF, outer, 0)

    for b in range(NBUF):
        row = base + (N_STEPS - NBUF + b) * CB
        pltpu.make_async_copy(rows[b], out_hbm.at[pl.ds(row, CB)],
                              ssem[b]).wait()


_mesh = plsc.VectorSubcoreMesh(
    core_axis_name="c", subcore_axis_name="s", num_cores=NC, num_subcores=NS
)

_embed = pl.kernel(
    _body,
    out_type=jax.ShapeDtypeStruct((B, S, D_MODEL), jnp.float32),
    mesh=_mesh,
    scratch_types=[
        pltpu.VMEM((CB, S), jnp.int32),
        pltpu.VMEM((CB, S), jnp.int32),
        pltpu.VMEM((CB, S, D_MODEL), jnp.float32),
        pltpu.VMEM((CB, S, D_MODEL), jnp.float32),
        pltpu.SemaphoreType.DMA,
        pltpu.SemaphoreType.DMA,
        pltpu.SemaphoreType.DMA,
        pltpu.SemaphoreType.DMA,
    ],
    compiler_params=pltpu.CompilerParams(use_tc_tiling_on_sc=False),
)


def kernel(word_indices, vocabulary):
    return _embed(word_indices.astype(jnp.int32), vocabulary)
